# baseline (speedup 1.0000x reference)
import functools

import jax
import jax.numpy as jnp
from jax import lax
from jax.experimental import pallas as pl
from jax.experimental.pallas import tpu as pltpu
from jax.experimental.pallas import tpu_sc as plsc

_NC = 2
_NS = 16
_NW = _NC * _NS

_G = 128
_K = 8


@functools.partial(jax.jit, static_argnames=())
def _embed_gather(x2d, table):
    n_groups, g = x2d.shape
    d = table.shape[1]
    groups_per_w = n_groups // _NW
    steps = groups_per_w // _K

    mesh = plsc.VectorSubcoreMesh(
        core_axis_name="c", subcore_axis_name="s",
        num_cores=_NC, num_subcores=_NS)

    @functools.partial(
        pl.kernel,
        out_type=jax.ShapeDtypeStruct((n_groups, g, d), jnp.float32),
        mesh=mesh,
        scratch_types=[
            pltpu.VMEM((_K, g), jnp.int32),
            pltpu.VMEM((_K, g, d), jnp.float32),
            pltpu.SemaphoreType.DMA,
        ],
        compiler_params=pltpu.CompilerParams(use_tc_tiling_on_sc=False),
    )
    def body(x_hbm, tab_hbm, out_hbm, idx_v, rows_v, gsem):
        wid = lax.axis_index("s") * _NC + lax.axis_index("c")
        g0 = wid * groups_per_w

        @pl.loop(0, steps)
        def _step(i):
            grp = g0 + i * _K
            pltpu.sync_copy(x_hbm.at[pl.ds(grp, _K)], idx_v)
            descs = [
                pltpu.async_copy(tab_hbm.at[idx_v.at[j]], rows_v.at[j], gsem)
                for j in range(_K)
            ]
            for dsc in descs:
                dsc.wait()
            pltpu.sync_copy(rows_v, out_hbm.at[pl.ds(grp, _K)])

    return body(x2d, table)


def kernel(x, embed_weights):
    b, h = x.shape
    d = embed_weights.shape[1]
    x2d = x.reshape(-1, _G).astype(jnp.int32)
    out = _embed_gather(x2d, embed_weights)
    return out.reshape(b, h, d)


# speedup vs baseline: 1.0184x; 1.0184x over previous
import functools

import jax
import jax.numpy as jnp
from jax import lax
from jax.experimental import pallas as pl
from jax.experimental.pallas import tpu as pltpu
from jax.experimental.pallas import tpu_sc as plsc

_NC = 2
_NS = 16
_NW = _NC * _NS

_G = 128
_K = 5


def _embed_gather(x2d, table):
    n_groups, g = x2d.shape
    d = table.shape[1]
    groups_per_w = n_groups // _NW
    steps = groups_per_w // _K

    mesh = plsc.VectorSubcoreMesh(
        core_axis_name="c", subcore_axis_name="s",
        num_cores=_NC, num_subcores=_NS)

    @functools.partial(
        pl.kernel,
        out_type=jax.ShapeDtypeStruct((n_groups, g, d), jnp.float32),
        mesh=mesh,
        scratch_types=[
            pltpu.VMEM((groups_per_w, g), jnp.int32),
            pltpu.VMEM((_K, g, d), jnp.float32),
            pltpu.VMEM((_K, g, d), jnp.float32),
            pltpu.SemaphoreType.DMA,
            pltpu.SemaphoreType.DMA,
            pltpu.SemaphoreType.DMA,
            pltpu.SemaphoreType.DMA,
        ],
        compiler_params=pltpu.CompilerParams(use_tc_tiling_on_sc=False),
    )
    def body(x_hbm, tab_hbm, out_hbm, idx_all, rows0, rows1,
             gsem0, gsem1, osem0, osem1):
        rows = (rows0, rows1)
        gsem = (gsem0, gsem1)
        osem = (osem0, osem1)
        wid = lax.axis_index("s") * _NC + lax.axis_index("c")
        gbase = wid * groups_per_w

        pltpu.sync_copy(x_hbm.at[pl.ds(gbase, groups_per_w)], idx_all)

        def fire(s, b):
            for j in range(_K):
                pltpu.async_copy(
                    tab_hbm.at[idx_all.at[s * _K + j]], rows[b].at[j],
                    gsem[b])

        def drain_g(b):
            pltpu.make_async_copy(
                out_hbm.at[pl.ds(0, _K)], rows[b], gsem[b]).wait()

        def start_out(s, b):
            pltpu.async_copy(
                rows[b], out_hbm.at[pl.ds(gbase + s * _K, _K)], osem[b])

        def drain_o(b):
            pltpu.make_async_copy(
                rows[b], out_hbm.at[pl.ds(0, _K)], osem[b]).wait()

        fire(0, 0)

        @pl.loop(0, steps, step=2)
        def _pair(i):
            @pl.when(i > 0)
            def _():
                drain_o(1)
            fire(i + 1, 1)
            drain_g(0)
            start_out(i, 0)

            @pl.when(i + 2 < steps)
            def _():
                drain_o(0)
                fire(i + 2, 0)
            drain_g(1)
            start_out(i + 1, 1)

        drain_o(0)
        drain_o(1)

    return body(x2d, table)


def kernel(x, embed_weights):
    b, h = x.shape
    d = embed_weights.shape[1]
    x2d = x.reshape(-1, _G).astype(jnp.int32)
    out = _embed_gather(x2d, embed_weights)
    return out.reshape(b, h, d)
